# baseline (speedup 1.0000x reference)
import functools

import jax
import jax.numpy as jnp
from jax import lax
from jax.experimental import pallas as pl
from jax.experimental.pallas import tpu as pltpu
from jax.experimental.pallas import tpu_sc as plsc

_DIM = 64
_B = 4096 * 200
_NC = 2
_NS = 16
_NW = _NC * _NS
_BPW = _B // _NW
_CHUNK = 512
_NCHUNK = _BPW // _CHUNK

_mesh = plsc.VectorSubcoreMesh(core_axis_name="c", subcore_axis_name="s")


@functools.partial(
    pl.kernel,
    mesh=_mesh,
    out_type=jax.ShapeDtypeStruct((_B, _DIM), jnp.float32),
    scratch_types=[
        pltpu.VMEM((_CHUNK,), jnp.int32),
        pltpu.VMEM((_CHUNK, _DIM), jnp.float32),
        pltpu.SemaphoreType.DMA,
    ],
    compiler_params=pltpu.CompilerParams(use_tc_tiling_on_sc=False),
)
def _emb_lookup(idx_hbm, table_hbm, out_hbm, idx_v, rows_v, sem):
    wid = lax.axis_index("s") * _NC + lax.axis_index("c")
    base = wid * _BPW

    def body(c, carry):
        off = base + c * _CHUNK
        pltpu.sync_copy(idx_hbm.at[pl.ds(off, _CHUNK)], idx_v)
        pltpu.async_copy(table_hbm.at[idx_v], rows_v, sem).wait()
        pltpu.sync_copy(rows_v, out_hbm.at[pl.ds(off, _CHUNK)])
        return carry

    lax.fori_loop(0, _NCHUNK, body, 0)


def kernel(x, table):
    idx = x.reshape(-1)
    out = _emb_lookup(idx, table)
    return out.reshape(x.shape + (table.shape[1],))


# speedup vs baseline: 1.0433x; 1.0433x over previous
import functools

import jax
import jax.numpy as jnp
from jax import lax
from jax.experimental import pallas as pl
from jax.experimental.pallas import tpu as pltpu
from jax.experimental.pallas import tpu_sc as plsc

_DIM = 64
_B = 4096 * 200
_NC = 2
_NS = 16
_NW = _NC * _NS
_BPW = _B // _NW
_CHUNK = 640
_NCHUNK = _BPW // _CHUNK

_mesh = plsc.VectorSubcoreMesh(core_axis_name="c", subcore_axis_name="s")


@functools.partial(
    pl.kernel,
    mesh=_mesh,
    out_type=jax.ShapeDtypeStruct((_B, _DIM), jnp.float32),
    scratch_types=[
        pltpu.VMEM((_BPW,), jnp.int32),
        pltpu.VMEM((2, _CHUNK, _DIM), jnp.float32),
        pltpu.SemaphoreType.DMA,
        pltpu.SemaphoreType.DMA,
        pltpu.SemaphoreType.DMA,
        pltpu.SemaphoreType.DMA,
    ],
    compiler_params=pltpu.CompilerParams(use_tc_tiling_on_sc=False),
)
def _emb_lookup(idx_hbm, table_hbm, out_hbm, idx_v, rows_v, g0, g1, o0, o1):
    wid = lax.axis_index("s") * _NC + lax.axis_index("c")
    base = wid * _BPW
    pltpu.sync_copy(idx_hbm.at[pl.ds(base, _BPW)], idx_v)

    gsem = (g0, g1)
    osem = (o0, o1)

    def idx_slice(c):
        return idx_v.at[pl.ds(c * _CHUNK, _CHUNK)]

    def out_slice(c):
        return out_hbm.at[pl.ds(base + c * _CHUNK, _CHUNK)]

    def start_gather(c, b):
        pltpu.async_copy(table_hbm.at[idx_slice(c)], rows_v.at[b], gsem[b])

    def wait_gather(c, b):
        pltpu.make_async_copy(table_hbm.at[idx_slice(c)], rows_v.at[b], gsem[b]).wait()

    def start_write(c, b):
        pltpu.async_copy(rows_v.at[b], out_slice(c), osem[b])

    def wait_write(c, b):
        pltpu.make_async_copy(rows_v.at[b], out_slice(c), osem[b]).wait()

    start_gather(0, 0)
    start_gather(1, 1)

    def body(i, carry):
        for b in (0, 1):
            cc = 2 * i + b
            wait_gather(cc, b)
            start_write(cc, b)
            wait_write(cc, b)
            start_gather(cc + 2, b)
        return carry

    lax.fori_loop(0, _NCHUNK // 2 - 1, body, 0)

    for b in (0, 1):
        wait_gather(_NCHUNK - 2 + b, b)
        start_write(_NCHUNK - 2 + b, b)
    for b in (0, 1):
        wait_write(_NCHUNK - 2 + b, b)


def kernel(x, table):
    idx = x.reshape(-1)
    out = _emb_lookup(idx, table)
    return out.reshape(x.shape + (table.shape[1],))


# speedup vs baseline: 1.2786x; 1.2255x over previous
import functools

import jax
import jax.numpy as jnp
from jax import lax
from jax.experimental import pallas as pl
from jax.experimental.pallas import tpu as pltpu
from jax.experimental.pallas import tpu_sc as plsc

_DIM = 64
_PAD = 128
_B = 4096 * 200
_NC = 2
_NS = 16
_NW = _NC * _NS
_BPW = _B // _NW
_CHUNK = 320
_NCHUNK = _BPW // _CHUNK

_mesh = plsc.VectorSubcoreMesh(core_axis_name="c", subcore_axis_name="s")


@functools.partial(
    pl.kernel,
    mesh=_mesh,
    out_type=jax.ShapeDtypeStruct((_B, _PAD), jnp.float32),
    scratch_types=[
        pltpu.VMEM((_BPW,), jnp.int32),
        pltpu.VMEM((2, _CHUNK, _PAD), jnp.float32),
        pltpu.SemaphoreType.DMA,
        pltpu.SemaphoreType.DMA,
        pltpu.SemaphoreType.DMA,
        pltpu.SemaphoreType.DMA,
    ],
)
def _emb_lookup(idx_hbm, table_hbm, out_hbm, idx_v, rows_v, g0, g1, o0, o1):
    wid = lax.axis_index("s") * _NC + lax.axis_index("c")
    base = wid * _BPW
    pltpu.sync_copy(idx_hbm.at[pl.ds(base, _BPW)], idx_v)

    gsem = (g0, g1)
    osem = (o0, o1)

    def idx_slice(c):
        return idx_v.at[pl.ds(c * _CHUNK, _CHUNK)]

    def out_slice(c):
        return out_hbm.at[pl.ds(base + c * _CHUNK, _CHUNK)]

    def start_gather(c, b):
        pltpu.async_copy(table_hbm.at[idx_slice(c)], rows_v.at[b], gsem[b])

    def wait_gather(c, b):
        pltpu.make_async_copy(table_hbm.at[idx_slice(c)], rows_v.at[b], gsem[b]).wait()

    def start_write(c, b):
        pltpu.async_copy(rows_v.at[b], out_slice(c), osem[b])

    def wait_write(c, b):
        pltpu.make_async_copy(rows_v.at[b], out_slice(c), osem[b]).wait()

    start_gather(0, 0)
    start_gather(1, 1)

    def body(i, carry):
        for b in (0, 1):
            cc = 2 * i + b
            wait_gather(cc, b)
            start_write(cc, b)
            wait_write(cc, b)
            start_gather(cc + 2, b)
        return carry

    lax.fori_loop(0, _NCHUNK // 2 - 1, body, 0)

    for b in (0, 1):
        wait_gather(_NCHUNK - 2 + b, b)
        start_write(_NCHUNK - 2 + b, b)
    for b in (0, 1):
        wait_write(_NCHUNK - 2 + b, b)


def kernel(x, table):
    tp = jnp.pad(table, ((0, 0), (0, _PAD - _DIM)))
    idx = x.reshape(-1)
    out = _emb_lookup(idx, tp)
    return out[:, :_DIM].reshape(x.shape + (table.shape[1],))
